# baseline (speedup 1.0000x reference)
import functools

import jax
import jax.numpy as jnp
from jax import lax
from jax.experimental import pallas as pl
from jax.experimental.pallas import tpu as pltpu
from jax.experimental.pallas import tpu_sc as plsc

M = 100000
D = 64
B = 16384

NS = 16
L = 16
C = 12800
NCHUNKS = 8
TAIL = M - (NCHUNKS - 1) * C
RPT_FULL = C // NS
RPT_TAIL = (TAIL // NS) // 8 * 8
TAIL_REM = TAIL - NS * RPT_TAIL
TRASH = C
BPT = B // NS
IDXW = 128
NXFER = BPT // IDXW

_mesh = plsc.VectorSubcoreMesh(core_axis_name="c", subcore_axis_name="s")


@functools.partial(
    pl.kernel,
    out_type=jax.ShapeDtypeStruct((M, D), jnp.float32),
    mesh=_mesh,
    compiler_params=pltpu.CompilerParams(use_tc_tiling_on_sc=False),
    scratch_types=[
        pltpu.VMEM((BPT,), jnp.int32),
        pltpu.VMEM((BPT, D), jnp.float32),
        pltpu.VMEM((NXFER, IDXW), jnp.int32),
        pltpu.VMEM_SHARED((C + 8, D), jnp.float32),
    ],
)
def _scatter_add_sc(mem_hbm, idx_hbm, val_hbm, out_hbm,
                    idx_v, val_v, lrow_v, accum):
    c = lax.axis_index("c")
    s = lax.axis_index("s")

    pltpu.sync_copy(idx_hbm.at[pl.ds(s * BPT, BPT)], idx_v)
    pltpu.sync_copy(val_hbm.at[pl.ds(s * BPT, BPT)], val_v)

    for k in range(NCHUNKS // 2):
        chunk = c + 2 * k
        lo = chunk * C
        hi = jnp.minimum(lo + C, M)

        def copy_in(rpt, lo=lo):
            pltpu.sync_copy(mem_hbm.at[pl.ds(lo + s * rpt, rpt)],
                            accum.at[pl.ds(s * rpt, rpt)])

        def copy_in_rem(lo=lo):
            pltpu.sync_copy(mem_hbm.at[pl.ds(lo + NS * RPT_TAIL, TAIL_REM)],
                            accum.at[pl.ds(NS * RPT_TAIL, TAIL_REM)])

        def copy_out(rpt, lo=lo):
            pltpu.sync_copy(accum.at[pl.ds(s * rpt, rpt)],
                            out_hbm.at[pl.ds(lo + s * rpt, rpt)])

        def copy_out_rem(lo=lo):
            pltpu.sync_copy(accum.at[pl.ds(NS * RPT_TAIL, TAIL_REM)],
                            out_hbm.at[pl.ds(lo + NS * RPT_TAIL, TAIL_REM)])

        tail_round = (k == NCHUNKS // 2 - 1)
        if not tail_round:
            copy_in(RPT_FULL)
        else:
            @pl.when(c == 0)
            def _():
                copy_in(RPT_FULL)

            @pl.when(c == 1)
            def _():
                copy_in(RPT_TAIL)

            @pl.when((c == 1) & (s == NS - 1))
            def _():
                copy_in_rem()

        plsc.subcore_barrier()

        for r in range(NXFER):
            for g in range(IDXW // L):
                v = idx_v[pl.ds(r * IDXW + g * L, L)]
                inr = (v >= lo) & (v < hi)
                lrow_v[r, pl.ds(g * L, L)] = jnp.where(inr, v - lo, TRASH)

        for r in range(NXFER):
            pltpu.sync_copy(val_v.at[pl.ds(r * IDXW, IDXW)],
                            accum.at[lrow_v.at[r]],
                            add=True)

        plsc.subcore_barrier()

        if not tail_round:
            copy_out(RPT_FULL)
        else:
            @pl.when(c == 0)
            def _():
                copy_out(RPT_FULL)

            @pl.when(c == 1)
            def _():
                copy_out(RPT_TAIL)

            @pl.when((c == 1) & (s == NS - 1))
            def _():
                copy_out_rem()

        plsc.subcore_barrier()


def kernel(mem, idx, val):
    return _scatter_add_sc(mem, idx.astype(jnp.int32), val)
